# baseline (speedup 1.0000x reference)
import functools

import jax
import jax.numpy as jnp
from jax import lax
from jax.experimental import pallas as pl
from jax.experimental.pallas import tpu as pltpu
from jax.experimental.pallas import tpu_sc as plsc

N = 200000
C = 64
B = 4
NPB = 50000
G = 32
CG = C // G
K = 27
CK = C * K
TILE = 1000
TPB = NPB // TILE
NT = N // TILE + 1
NP_ROWS = NT * TILE

STILE = 2000
SPB = NPB // STILE

NC = 2
NS = 16
NW = NC * NS
CH = 125
NR = N // CH
PER_W = N // NW
ROUNDS = PER_W // CH
P = 1280
GR = 256
KP32 = 32
SWORDS = CH * KP32
POFF = SWORDS
STRIDE = SWORDS + 128
GIDX_CAP = 3392
ZIDX = N * K
PERM = ([2 * i for i in range(16)] + [2 * i + 1 for i in range(16)]
        + [32 + 2 * i for i in range(16)]
        + [33 + 2 * i for i in range(16)])
INVPERM = [0] * C
for _i, _p in enumerate(PERM):
    INVPERM[_p] = _i


def _stats_body(x_ref, s_ref, q_ref):
    t = pl.program_id(1)
    x = x_ref[...]
    s = jnp.sum(x, axis=0)[None, None, :]
    q = jnp.sum(x * x, axis=0)[None, None, :]

    @pl.when(t == 0)
    def _init():
        s_ref[...] = s
        q_ref[...] = q

    @pl.when(t != 0)
    def _acc():
        s_ref[...] += s
        q_ref[...] += q


_stats_call = pl.pallas_call(
    _stats_body,
    grid=(B, SPB),
    in_specs=[pl.BlockSpec((STILE, C), lambda b, t: (b * SPB + t, 0))],
    out_specs=[
        pl.BlockSpec((1, 1, C), lambda b, t: (b, 0, 0)),
        pl.BlockSpec((1, 1, C), lambda b, t: (b, 0, 0)),
    ],
    out_shape=[
        jax.ShapeDtypeStruct((B, 1, C), jnp.float32),
        jax.ShapeDtypeStruct((B, 1, C), jnp.float32),
    ],
)


def _matnorm_body(x_ref, a_ref, c_ref, w_ref, y_ref):
    t = pl.program_id(0)
    x = x_ref[...]
    a = a_ref[0]
    cc = c_ref[0]
    h = x * a + cc
    h = h * jax.nn.sigmoid(h)
    row = t * TILE + lax.broadcasted_iota(jnp.int32, (TILE, 1), 0)
    h = jnp.where(row < N, h, 0.0)
    y_ref[...] = jnp.dot(
        h, w_ref[...], preferred_element_type=jnp.float32
    ).astype(jnp.bfloat16)


_matnorm_call = pl.pallas_call(
    _matnorm_body,
    grid=(NT,),
    in_specs=[
        pl.BlockSpec((TILE, C), lambda t: (jnp.minimum(t, N // TILE - 1), 0)),
        pl.BlockSpec((1, 1, C), lambda t: (jnp.minimum(t // TPB, B - 1), 0, 0)),
        pl.BlockSpec((1, 1, C), lambda t: (jnp.minimum(t // TPB, B - 1), 0, 0)),
        pl.BlockSpec((C, CK), lambda t: (0, 0)),
    ],
    out_specs=pl.BlockSpec((TILE, CK), lambda t: (t, 0)),
    out_shape=jax.ShapeDtypeStruct((NP_ROWS, CK), jnp.bfloat16),
)


def _emb_body(e_ref, w_ref, o_ref):
    e = e_ref[...]
    h = e * jax.nn.sigmoid(e)
    o_ref[...] = jnp.dot(h, w_ref[...], preferred_element_type=jnp.float32)


def _emb_call(emb, We):
    return pl.pallas_call(
        _emb_body,
        out_shape=jax.ShapeDtypeStruct((B, C), jnp.float32),
    )(emb, We)


def _make_gather():
    mesh = plsc.VectorSubcoreMesh(
        core_axis_name="c", subcore_axis_name="s", num_cores=NC,
        num_subcores=NS)

    scratch = [
        pltpu.VMEM((STRIDE,), jnp.int32),
        pltpu.VMEM((GIDX_CAP,), jnp.int32),
        pltpu.VMEM((P, C), jnp.bfloat16),
        pltpu.VMEM((CH, C), jnp.float32),
        pltpu.VMEM((C,), jnp.float32),
        pltpu.SemaphoreType.DMA,
    ]

    def body(tab_ref, im_ref, bias_ref, *rest):
        (out_ref, im_v, gidx_v, rows_v, out_v, bias_v, sem) = rest
        cid = lax.axis_index("c")
        sid = lax.axis_index("s")
        wid = sid * NC + cid
        pltpu.sync_copy(bias_ref, bias_v)
        zvec = jnp.full((16,), ZIDX, jnp.int32)

        @pl.loop(0, jnp.int32(GIDX_CAP // 16))
        def _init(i):
            gidx_v[pl.ds(pl.multiple_of(i * 16, 16), 16)] = zvec

        @pl.loop(0, jnp.int32(ROUNDS))
        def _round(r):
            rb = wid * ROUNDS + r
            vbase = pl.multiple_of(rb * CH, CH)
            pltpu.sync_copy(im_ref.at[rb], im_v)

            @pl.loop(0, jnp.int32(SWORDS // 16), init_carry=jnp.int32(0))
            def _compact(i, off):
                vec = im_v[pl.ds(pl.multiple_of(i * 16, 16), 16)]
                msk = vec < zvec
                plsc.store_compressed(gidx_v.at[pl.ds(off, 16)], vec,
                                      mask=msk)
                return off + plsc.all_reduce_population_count(msk)[0]

            rcnt = _compact

            @pl.loop(0, jnp.int32(CH))
            def _initout(q):
                for ci in range(C // 16):
                    sl = pl.ds(ci * 16, 16)
                    out_v[q, sl] = bias_v[sl]

            nwv = (rcnt + P - 1) // P

            @pl.loop(0, nwv)
            def _wave(w):
                wbase = pl.multiple_of(w * P, P)
                wcnt = jnp.minimum(rcnt - wbase, P)
                gw = (wcnt + GR - 1) // GR

                @pl.loop(0, gw)
                def _fire(t):
                    toff = pl.multiple_of(wbase + t * GR, GR)
                    roff = pl.multiple_of(t * GR, GR)
                    pltpu.async_copy(
                        tab_ref.at[gidx_v.at[pl.ds(toff, GR)]],
                        rows_v.at[pl.ds(roff, GR)], sem)

                @pl.loop(0, gw)
                def _drain(t):
                    toff = pl.multiple_of(wbase + t * GR, GR)
                    roff = pl.multiple_of(t * GR, GR)
                    pltpu.make_async_copy(
                        tab_ref.at[gidx_v.at[pl.ds(toff, GR)]],
                        rows_v.at[pl.ds(roff, GR)], sem).wait()

                @pl.loop(0, jnp.int32(CH), init_carry=jnp.int32(0))
                def _vox(q, p):
                    cq = im_v[pl.ds(POFF + q, 16)][0]
                    i0 = jnp.clip(wbase - p, 0, cq)
                    i1 = jnp.clip(wbase + P - p, 0, cq)
                    accs = tuple(
                        out_v[q, pl.ds(ci * 16, 16)]
                        for ci in range(C // 16))

                    def _pair(i, accs):
                        row = p + i - wbase
                        new = []
                        for cb in range(C // 32):
                            ab = rows_v[row, pl.ds(cb * 32, 32)]
                            ev, od = plsc.unpack(
                                ab, format=plsc.PackFormat.INTERLEAVED,
                                preferred_element_type=jnp.float32)
                            new.append(accs[2 * cb] + ev)
                            new.append(accs[2 * cb + 1] + od)
                        return tuple(new)

                    accs = lax.fori_loop(i0, i1, _pair, accs)
                    for ci in range(C // 16):
                        out_v[q, pl.ds(ci * 16, 16)] = accs[ci]
                    return p + cq

            pltpu.sync_copy(out_v, out_ref.at[pl.ds(vbase, CH)])

    return pl.kernel(
        body,
        out_type=jax.ShapeDtypeStruct((N, C), jnp.float32),
        mesh=mesh,
        scratch_types=scratch,
        compiler_params=pltpu.CompilerParams(
            use_tc_tiling_on_sc=False, needs_layout_passes=False),
    )


_gather_call = _make_gather()


def _coeffs(s_c, q_c, gamma, beta):
    cnt = float(NPB * CG)
    sg = s_c.reshape(B, G, CG).sum(-1)
    qg = q_c.reshape(B, G, CG).sum(-1)
    mean_g = sg / cnt
    var_g = qg / cnt - mean_g * mean_g
    rstd_g = lax.rsqrt(var_g + 1e-5)
    mean_c = jnp.repeat(mean_g, CG, axis=1)
    rstd_c = jnp.repeat(rstd_g, CG, axis=1)
    a = rstd_c * gamma[None, :]
    c = beta[None, :] - mean_c * a
    return a, c


def kernel(feats, emb, gamma1, beta1, W1, b1c, We, be, gamma2, beta2, W2,
           b2c, batch_idx, nbrs):
    del batch_idx

    karr = jnp.arange(K, dtype=jnp.int32)[None, :]
    nbrsT = nbrs.astype(jnp.int32).T
    flat = nbrsT * K + karr
    cnt = (nbrsT < N).sum(1, dtype=jnp.int32)
    flatp = jnp.concatenate(
        [flat, jnp.full((N, KP32 - K), ZIDX, jnp.int32)], axis=1)
    im = jnp.concatenate([
        flatp.reshape(NR, SWORDS),
        cnt.reshape(NR, CH),
        jnp.zeros((NR, STRIDE - SWORDS - CH), jnp.int32),
    ], axis=1)

    Wcat1 = jnp.transpose(W1, (1, 0, 2)).reshape(C, CK)
    Wcat2 = jnp.transpose(W2, (1, 0, 2)).reshape(C, CK)

    permj = jnp.array(PERM, dtype=jnp.int32)
    invj = jnp.array(INVPERM, dtype=jnp.int32)

    s1, q1 = _stats_call(feats)
    a1, c1 = _coeffs(s1[:, 0, :], q1[:, 0, :], gamma1, beta1)
    Y1 = _matnorm_call(feats, a1[:, None, :], c1[:, None, :], Wcat1)
    h_perm = _gather_call(Y1.reshape(NP_ROWS * K, C), im,
                          jnp.zeros((C,), jnp.float32))

    emb_out = _emb_call(emb, We) + be[None, :]
    t_bc = emb_out + b1c[None, :]

    s2, q2 = _stats_call(h_perm)
    s2c = s2[:, 0, :][:, invj]
    q2c = q2[:, 0, :][:, invj]
    s2s = s2c + NPB * t_bc
    q2s = q2c + 2.0 * t_bc * s2c + NPB * t_bc * t_bc
    a2, c2b = _coeffs(s2s, q2s, gamma2, beta2)
    c2 = c2b + t_bc * a2

    Y2 = _matnorm_call(h_perm, a2[:, permj][:, None, :],
                       c2[:, permj][:, None, :], Wcat2[permj])
    out_perm = _gather_call(Y2.reshape(NP_ROWS * K, C), im, b2c[permj])
    return feats + out_perm[:, invj]


# speedup vs baseline: 1.3720x; 1.3720x over previous
import functools

import jax
import jax.numpy as jnp
from jax import lax
from jax.experimental import pallas as pl
from jax.experimental.pallas import tpu as pltpu
from jax.experimental.pallas import tpu_sc as plsc

N = 200000
C = 64
B = 4
NPB = 50000
G = 32
CG = C // G
K = 27
CK = C * K
TILE = 1000
TPB = NPB // TILE
NT = N // TILE + 1
NP_ROWS = NT * TILE

STILE = 2000
SPB = NPB // STILE

NC = 2
NS = 16
NW = NC * NS
CH = 125
NR = N // CH
PER_W = N // NW
ROUNDS = PER_W // CH
P = 1280
GR = 256
KP32 = 32
SWORDS = CH * KP32
POFF = SWORDS
STRIDE = SWORDS + 128
GIDX_CAP = 3392
ZIDX = N * K


def _stats_body(x_ref, s_ref, q_ref):
    t = pl.program_id(1)
    x = x_ref[...]
    s = jnp.sum(x, axis=0)[None, None, :]
    q = jnp.sum(x * x, axis=0)[None, None, :]

    @pl.when(t == 0)
    def _init():
        s_ref[...] = s
        q_ref[...] = q

    @pl.when(t != 0)
    def _acc():
        s_ref[...] += s
        q_ref[...] += q


_stats_call = pl.pallas_call(
    _stats_body,
    grid=(B, SPB),
    in_specs=[pl.BlockSpec((STILE, C), lambda b, t: (b * SPB + t, 0))],
    out_specs=[
        pl.BlockSpec((1, 1, C), lambda b, t: (b, 0, 0)),
        pl.BlockSpec((1, 1, C), lambda b, t: (b, 0, 0)),
    ],
    out_shape=[
        jax.ShapeDtypeStruct((B, 1, C), jnp.float32),
        jax.ShapeDtypeStruct((B, 1, C), jnp.float32),
    ],
)


def _matnorm_body(x_ref, a_ref, c_ref, w_ref, y_ref):
    t = pl.program_id(0)
    x = x_ref[...]
    a = a_ref[0]
    cc = c_ref[0]
    h = x * a + cc
    h = h * jax.nn.sigmoid(h)
    row = t * TILE + lax.broadcasted_iota(jnp.int32, (TILE, 1), 0)
    h = jnp.where(row < N, h, 0.0)
    y_ref[...] = jnp.dot(h, w_ref[...], preferred_element_type=jnp.float32)


_matnorm_call = pl.pallas_call(
    _matnorm_body,
    grid=(NT,),
    in_specs=[
        pl.BlockSpec((TILE, C), lambda t: (jnp.minimum(t, N // TILE - 1), 0)),
        pl.BlockSpec((1, 1, C), lambda t: (jnp.minimum(t // TPB, B - 1), 0, 0)),
        pl.BlockSpec((1, 1, C), lambda t: (jnp.minimum(t // TPB, B - 1), 0, 0)),
        pl.BlockSpec((C, CK), lambda t: (0, 0)),
    ],
    out_specs=pl.BlockSpec((TILE, CK), lambda t: (t, 0)),
    out_shape=jax.ShapeDtypeStruct((NP_ROWS, CK), jnp.float32),
)


def _emb_body(e_ref, w_ref, o_ref):
    e = e_ref[...]
    h = e * jax.nn.sigmoid(e)
    o_ref[...] = jnp.dot(h, w_ref[...], preferred_element_type=jnp.float32)


def _emb_call(emb, We):
    return pl.pallas_call(
        _emb_body,
        out_shape=jax.ShapeDtypeStruct((B, C), jnp.float32),
    )(emb, We)


def _make_gather(residual: bool):
    mesh = plsc.VectorSubcoreMesh(
        core_axis_name="c", subcore_axis_name="s", num_cores=NC,
        num_subcores=NS)

    scratch = [
        pltpu.VMEM((STRIDE,), jnp.int32),
        pltpu.VMEM((GIDX_CAP,), jnp.int32),
        pltpu.VMEM((P, C), jnp.float32),
        pltpu.VMEM((CH, C), jnp.float32),
        pltpu.VMEM((C,), jnp.float32),
        pltpu.VMEM((CH, C), jnp.float32),
        pltpu.SemaphoreType.DMA,
    ]

    def body(tab_ref, im_ref, bias_ref, *rest):
        if residual:
            (res_ref, out_ref, im_v, gidx_v, rows_v, out_v, bias_v, res_v,
             sem) = rest
        else:
            (out_ref, im_v, gidx_v, rows_v, out_v, bias_v, res_v,
             sem) = rest
        cid = lax.axis_index("c")
        sid = lax.axis_index("s")
        wid = sid * NC + cid
        pltpu.sync_copy(bias_ref, bias_v)
        zvec = jnp.full((16,), ZIDX, jnp.int32)

        @pl.loop(0, jnp.int32(GIDX_CAP // 16))
        def _init(i):
            gidx_v[pl.ds(pl.multiple_of(i * 16, 16), 16)] = zvec

        @pl.loop(0, jnp.int32(ROUNDS))
        def _round(r):
            rb = wid * ROUNDS + r
            vbase = pl.multiple_of(rb * CH, CH)
            pltpu.sync_copy(im_ref.at[rb], im_v)
            if residual:
                pltpu.sync_copy(res_ref.at[pl.ds(vbase, CH)], res_v)

            @pl.loop(0, jnp.int32(SWORDS // 16), init_carry=jnp.int32(0))
            def _compact(i, off):
                vec = im_v[pl.ds(pl.multiple_of(i * 16, 16), 16)]
                msk = vec < zvec
                plsc.store_compressed(gidx_v.at[pl.ds(off, 16)], vec,
                                      mask=msk)
                return off + plsc.all_reduce_population_count(msk)[0]

            rcnt = _compact

            @pl.loop(0, jnp.int32(CH))
            def _initout(q):
                for ci in range(C // 16):
                    sl = pl.ds(ci * 16, 16)
                    a = bias_v[sl]
                    if residual:
                        a = a + res_v[q, sl]
                    out_v[q, sl] = a

            nwv = (rcnt + P - 1) // P

            @pl.loop(0, nwv)
            def _wave(w):
                wbase = pl.multiple_of(w * P, P)
                wcnt = jnp.minimum(rcnt - wbase, P)
                gw = (wcnt + GR - 1) // GR

                @pl.loop(0, gw)
                def _fire(t):
                    toff = pl.multiple_of(wbase + t * GR, GR)
                    roff = pl.multiple_of(t * GR, GR)
                    pltpu.async_copy(
                        tab_ref.at[gidx_v.at[pl.ds(toff, GR)]],
                        rows_v.at[pl.ds(roff, GR)], sem)

                @pl.loop(0, gw)
                def _drain(t):
                    toff = pl.multiple_of(wbase + t * GR, GR)
                    roff = pl.multiple_of(t * GR, GR)
                    pltpu.make_async_copy(
                        tab_ref.at[gidx_v.at[pl.ds(toff, GR)]],
                        rows_v.at[pl.ds(roff, GR)], sem).wait()

                @pl.loop(0, jnp.int32(CH), init_carry=jnp.int32(0))
                def _vox(q, p):
                    cq = im_v[pl.ds(POFF + q, 16)][0]
                    i0 = jnp.clip(wbase - p, 0, cq)
                    i1 = jnp.clip(wbase + P - p, 0, cq)
                    accs = tuple(
                        out_v[q, pl.ds(ci * 16, 16)]
                        for ci in range(C // 16))

                    def _pair(i, accs):
                        row = p + i - wbase
                        return tuple(
                            accs[ci] + rows_v[row, pl.ds(ci * 16, 16)]
                            for ci in range(C // 16))

                    accs = lax.fori_loop(i0, i1, _pair, accs)
                    for ci in range(C // 16):
                        out_v[q, pl.ds(ci * 16, 16)] = accs[ci]
                    return p + cq

            pltpu.sync_copy(out_v, out_ref.at[pl.ds(vbase, CH)])

    return pl.kernel(
        body,
        out_type=jax.ShapeDtypeStruct((N, C), jnp.float32),
        mesh=mesh,
        scratch_types=scratch,
        compiler_params=pltpu.CompilerParams(
            use_tc_tiling_on_sc=False, needs_layout_passes=False),
    )


_gather_plain = _make_gather(residual=False)
_gather_res = _make_gather(residual=True)


def _coeffs(s_c, q_c, gamma, beta):
    cnt = float(NPB * CG)
    sg = s_c.reshape(B, G, CG).sum(-1)
    qg = q_c.reshape(B, G, CG).sum(-1)
    mean_g = sg / cnt
    var_g = qg / cnt - mean_g * mean_g
    rstd_g = lax.rsqrt(var_g + 1e-5)
    mean_c = jnp.repeat(mean_g, CG, axis=1)
    rstd_c = jnp.repeat(rstd_g, CG, axis=1)
    a = rstd_c * gamma[None, :]
    c = beta[None, :] - mean_c * a
    return a, c


def kernel(feats, emb, gamma1, beta1, W1, b1c, We, be, gamma2, beta2, W2,
           b2c, batch_idx, nbrs):
    del batch_idx

    karr = jnp.arange(K, dtype=jnp.int32)[None, :]
    nbrsT = nbrs.astype(jnp.int32).T
    flat = nbrsT * K + karr
    cnt = (nbrsT < N).sum(1, dtype=jnp.int32)
    flatp = jnp.concatenate(
        [flat, jnp.full((N, KP32 - K), ZIDX, jnp.int32)], axis=1)
    im = jnp.concatenate([
        flatp.reshape(NR, SWORDS),
        cnt.reshape(NR, CH),
        jnp.zeros((NR, STRIDE - SWORDS - CH), jnp.int32),
    ], axis=1)

    Wcat1 = jnp.transpose(W1, (1, 0, 2)).reshape(C, CK)
    Wcat2 = jnp.transpose(W2, (1, 0, 2)).reshape(C, CK)

    s1, q1 = _stats_call(feats)
    a1, c1 = _coeffs(s1[:, 0, :], q1[:, 0, :], gamma1, beta1)
    Y1 = _matnorm_call(feats, a1[:, None, :], c1[:, None, :], Wcat1)
    h_raw = _gather_plain(Y1.reshape(NP_ROWS * K, C), im,
                          jnp.zeros((C,), jnp.float32))

    emb_out = _emb_call(emb, We) + be[None, :]
    t_bc = emb_out + b1c[None, :]

    s2, q2 = _stats_call(h_raw)
    s2c = s2[:, 0, :]
    q2c = q2[:, 0, :]
    s2s = s2c + NPB * t_bc
    q2s = q2c + 2.0 * t_bc * s2c + NPB * t_bc * t_bc
    a2, c2b = _coeffs(s2s, q2s, gamma2, beta2)
    c2 = c2b + t_bc * a2

    Y2 = _matnorm_call(h_raw, a2[:, None, :], c2[:, None, :], Wcat2)
    out = _gather_res(Y2.reshape(NP_ROWS * K, C), im, b2c, feats)
    return out
